# baseline (speedup 1.0000x reference)
import jax
import jax.numpy as jnp
from jax import lax
from jax.experimental import pallas as pl
from jax.experimental.pallas import tpu as pltpu
from jax.experimental.pallas import tpu_sc as plsc

B = 16384
D = 16
V = 100000
GROUP = 8
LINE = GROUP * D

_INFO = plsc.get_sparse_core_info()
_NC = _INFO.num_cores
_NS = _INFO.num_subcores
_NW = _NC * _NS
_BPW = B // _NW
_CH = 64
_NCHUNK = _BPW // _CH


def _body(contrib_lines, recip_lines, contrib_idx, recip_idx,
          xr_out, xc_out,
          idx_rv, idx_cv, tid_r, tid_c,
          rows_a, rows_b, out_a, out_b,
          sem_a, sem_b, sem_wa, sem_wb):
    wid = lax.axis_index("s") * _NC + lax.axis_index("c")
    base = wid * _BPW
    sl = pl.ds(base, _BPW)
    pltpu.sync_copy(recip_idx.at[sl], idx_rv)
    pltpu.sync_copy(contrib_idx.at[sl], idx_cv)

    def tids(k, _):
        s = pl.ds(k * 16, 16)
        tid_r[s] = lax.shift_right_logical(idx_rv[s], 3)
        tid_c[s] = lax.shift_right_logical(idx_cv[s], 3)
        return 0

    lax.fori_loop(0, _BPW // 16, tids, 0)

    lanes = lax.iota(jnp.int32, 16)

    def select(rows, idx_v, c0, out_v):
        def grp(g, _):
            offs = (idx_v[pl.ds(c0 + g * 16, 16)] & 7) * D
            jv = lanes + g * 16
            for l in range(D):
                vals = plsc.load_gather(rows, [jv, offs + l])
                plsc.store_scatter(out_v, [jv, lanes * 0 + l], vals)
            return 0

        lax.fori_loop(0, _CH // 16, grp, 0)

    steps = [(recip_lines, tid_r, idx_rv, xr_out, q) for q in range(_NCHUNK)]
    steps += [(contrib_lines, tid_c, idx_cv, xc_out, q) for q in range(_NCHUNK)]
    bufs = (rows_a, rows_b)
    sems = (sem_a, sem_b)
    obufs = (out_a, out_b)
    wsems = (sem_wa, sem_wb)

    def issue(k):
        lines, tid, _, _, q = steps[k]
        b = k % 2
        return pltpu.async_copy(
            lines.at[tid.at[pl.ds(q * _CH, _CH)]], bufs[b], sems[b])

    cp = [issue(0), issue(1)]
    wcp = [None, None]
    for k in range(2, len(steps) + 2):
        pk = k - 2
        b = pk % 2
        cp[b].wait()
        _, _, idx_v, out_hbm, q = steps[pk]
        if wcp[b] is not None:
            wcp[b].wait()
        select(bufs[b], idx_v, q * _CH, obufs[b])
        if k < len(steps):
            cp[b] = issue(k)
        wcp[b] = pltpu.async_copy(
            obufs[b], out_hbm.at[pl.ds(base + q * _CH, _CH)], wsems[b])
    wcp[0].wait()
    wcp[1].wait()


@jax.jit
def kernel(contrib_table, recip_table, contrib_idx, recip_idx):
    mesh = plsc.VectorSubcoreMesh(core_axis_name="c", subcore_axis_name="s")
    contrib_lines = contrib_table.reshape(V // GROUP, LINE)
    recip_lines = recip_table.reshape(V // GROUP, LINE)
    xr, xc = pl.kernel(
        _body,
        mesh=mesh,
        out_type=(
            jax.ShapeDtypeStruct((B, D), jnp.float32),
            jax.ShapeDtypeStruct((B, D), jnp.float32),
        ),
        scratch_types=[
            pltpu.VMEM((_BPW,), jnp.int32),
            pltpu.VMEM((_BPW,), jnp.int32),
            pltpu.VMEM((_BPW,), jnp.int32),
            pltpu.VMEM((_BPW,), jnp.int32),
            pltpu.VMEM((_CH, LINE), jnp.float32),
            pltpu.VMEM((_CH, LINE), jnp.float32),
            pltpu.VMEM((_CH, D), jnp.float32),
            pltpu.VMEM((_CH, D), jnp.float32),
            pltpu.SemaphoreType.DMA,
            pltpu.SemaphoreType.DMA,
            pltpu.SemaphoreType.DMA,
            pltpu.SemaphoreType.DMA,
        ],
        compiler_params=pltpu.CompilerParams(needs_layout_passes=False),
    )(contrib_lines, recip_lines, contrib_idx, recip_idx)
    return xr, xc


# speedup vs baseline: 1.3790x; 1.3790x over previous
import jax
import jax.numpy as jnp
from jax import lax
from jax.experimental import pallas as pl
from jax.experimental.pallas import tpu as pltpu
from jax.experimental.pallas import tpu_sc as plsc

B = 16384
D = 16
V = 100000

_INFO = plsc.get_sparse_core_info()
_NC = _INFO.num_cores
_NS = _INFO.num_subcores
_NW = _NC * _NS
_BPW = B // _NW


def _body(contrib_table, recip_table, contrib_idx, recip_idx,
          xr_out, xc_out,
          idx_rv, idx_cv, out_r, out_c,
          sem_gr, sem_gc, sem_wr, sem_wc):
    wid = lax.axis_index("s") * _NC + lax.axis_index("c")
    base = wid * _BPW
    sl = pl.ds(base, _BPW)
    pltpu.sync_copy(recip_idx.at[sl], idx_rv)
    pltpu.sync_copy(contrib_idx.at[sl], idx_cv)

    half = _BPW // 2

    def make_grp(h):
        def grp(g, _):
            j0 = h * half + g * 16
            o0 = g * 16
            vr = idx_rv[pl.ds(j0, 16)]
            vc = idx_cv[pl.ds(j0, 16)]
            for l in range(16):
                pltpu.async_copy(recip_table.at[pl.ds(vr[l], 1)],
                                 out_r.at[pl.ds(o0 + l, 1)], sem_gr)
                pltpu.async_copy(contrib_table.at[pl.ds(vc[l], 1)],
                                 out_c.at[pl.ds(o0 + l, 1)], sem_gc)
            return 0
        return grp

    wr = wc = None
    for h in range(2):
        if wr is not None:
            wr.wait()
            wc.wait()
        lax.fori_loop(0, half // 16, make_grp(h), 0)
        pltpu.make_async_copy(xr_out.at[pl.ds(base, half)],
                              out_r, sem_gr).wait()
        pltpu.make_async_copy(xc_out.at[pl.ds(base, half)],
                              out_c, sem_gc).wait()
        hs = pl.ds(base + h * half, half)
        wr = pltpu.async_copy(out_r, xr_out.at[hs], sem_wr)
        wc = pltpu.async_copy(out_c, xc_out.at[hs], sem_wc)
    wr.wait()
    wc.wait()


@jax.jit
def kernel(contrib_table, recip_table, contrib_idx, recip_idx):
    mesh = plsc.VectorSubcoreMesh(core_axis_name="c", subcore_axis_name="s")
    xr, xc = pl.kernel(
        _body,
        mesh=mesh,
        out_type=(
            jax.ShapeDtypeStruct((B, D), jnp.float32),
            jax.ShapeDtypeStruct((B, D), jnp.float32),
        ),
        scratch_types=[
            pltpu.VMEM((_BPW,), jnp.int32),
            pltpu.VMEM((_BPW,), jnp.int32),
            pltpu.VMEM((_BPW // 2, D), jnp.float32),
            pltpu.VMEM((_BPW // 2, D), jnp.float32),
            pltpu.SemaphoreType.DMA,
            pltpu.SemaphoreType.DMA,
            pltpu.SemaphoreType.DMA,
            pltpu.SemaphoreType.DMA,
        ],
        compiler_params=pltpu.CompilerParams(needs_layout_passes=False),
    )(contrib_table, recip_table, contrib_idx, recip_idx)
    return xr, xc


# speedup vs baseline: 1.4025x; 1.0171x over previous
import jax
import jax.numpy as jnp
from jax import lax
from jax.experimental import pallas as pl
from jax.experimental.pallas import tpu as pltpu
from jax.experimental.pallas import tpu_sc as plsc

B = 16384
D = 16
V = 100000

_INFO = plsc.get_sparse_core_info()
_NC = _INFO.num_cores
_NS = _INFO.num_subcores
_NW = _NC * _NS
_BPW = B // _NW
_Q = 128
_NQ = _BPW // _Q
_NSTEP = 2 * _NQ
_NBUF = 6


def _body(contrib_table, recip_table, contrib_idx, recip_idx,
          xr_out, xc_out,
          idx_rv, idx_cv,
          b0, b1, b2, b3, b4, b5,
          sem_ir, sem_ic, sem_gr, sem_gc, sem_wr, sem_wc):
    wid = lax.axis_index("s") * _NC + lax.axis_index("c")
    base = wid * _BPW
    sl = pl.ds(base, _BPW)
    ir = pltpu.async_copy(recip_idx.at[sl], idx_rv, sem_ir)
    ic = pltpu.async_copy(contrib_idx.at[sl], idx_cv, sem_ic)
    ir.wait()
    ic.wait()

    bufs = (b0, b1, b2, b3, b4, b5)
    tabs = (recip_table, contrib_table)
    idxs = (idx_rv, idx_cv)
    outs = (xr_out, xc_out)
    gsems = (sem_gr, sem_gc)
    wsems = (sem_wr, sem_wc)

    def issue(k):
        t, q, buf = k % 2, k // 2, bufs[k % _NBUF]
        tab, idx_v = tabs[t], idxs[t]

        def grp(g, _):
            j0 = q * _Q + g * 16
            v = idx_v[pl.ds(j0, 16)]
            for l in range(16):
                pltpu.async_copy(tab.at[pl.ds(v[l], 1)],
                                 buf.at[pl.ds(g * 16 + l, 1)], gsems[t])
            return 0

        lax.fori_loop(0, _Q // 16, grp, 0)

    def drain_write(k):
        t, q, buf = k % 2, k // 2, bufs[k % _NBUF]
        o = pl.ds(base + q * _Q, _Q)
        pltpu.make_async_copy(outs[t].at[o], buf, gsems[t]).wait()
        return pltpu.async_copy(buf, outs[t].at[o], wsems[t])

    wh = [None] * _NSTEP
    for k in range(_NSTEP):
        if k >= _NBUF:
            d = k - _NBUF
            wh[d] = drain_write(d)
            wh[d].wait()
        issue(k)
    for d in range(_NSTEP):
        if wh[d] is None:
            wh[d] = drain_write(d)
    for d in range(_NSTEP - _NBUF, _NSTEP):
        wh[d].wait()


@jax.jit
def kernel(contrib_table, recip_table, contrib_idx, recip_idx):
    mesh = plsc.VectorSubcoreMesh(core_axis_name="c", subcore_axis_name="s")
    xr, xc = pl.kernel(
        _body,
        mesh=mesh,
        out_type=(
            jax.ShapeDtypeStruct((B, D), jnp.float32),
            jax.ShapeDtypeStruct((B, D), jnp.float32),
        ),
        scratch_types=[
            pltpu.VMEM((_BPW,), jnp.int32),
            pltpu.VMEM((_BPW,), jnp.int32),
            pltpu.VMEM((_Q, D), jnp.float32),
            pltpu.VMEM((_Q, D), jnp.float32),
            pltpu.VMEM((_Q, D), jnp.float32),
            pltpu.VMEM((_Q, D), jnp.float32),
            pltpu.VMEM((_Q, D), jnp.float32),
            pltpu.VMEM((_Q, D), jnp.float32),
            pltpu.SemaphoreType.DMA,
            pltpu.SemaphoreType.DMA,
            pltpu.SemaphoreType.DMA,
            pltpu.SemaphoreType.DMA,
            pltpu.SemaphoreType.DMA,
            pltpu.SemaphoreType.DMA,
        ],
        compiler_params=pltpu.CompilerParams(needs_layout_passes=False),
    )(contrib_table, recip_table, contrib_idx, recip_idx)
    return xr, xc
